# baseline (speedup 1.0000x reference)
import functools

import jax
import jax.numpy as jnp
from jax import lax
from jax.experimental import pallas as pl
from jax.experimental.pallas import tpu as pltpu
from jax.experimental.pallas import tpu_sc as plsc

D_MODEL = 1024
VOCAB = 100000


_NC = 2
_NS = 16
_NW = _NC * _NS
_B = 4 * 4096
_BPW = _B // _NW
_C = 16
_NCHUNK = _BPW // _C


_NBUF = 4


_ROWS_PER_TOKROW = 4096 // _BPW


def _gather_body(table_hbm, idx_hbm, out_hbm, idx_v, rows, gsems, wsems):
    wid = lax.axis_index("s") * _NC + lax.axis_index("c")
    base = wid * _BPW
    pltpu.sync_copy(
        idx_hbm.at[wid // _ROWS_PER_TOKROW,
                   pl.ds((wid % _ROWS_PER_TOKROW) * _BPW, _BPW)], idx_v)

    def gather(c, b):
        pltpu.async_copy(
            table_hbm.at[idx_v.at[pl.ds(c * _C, _C)]], rows[b], gsems[b])

    def wait_gather(c, b):
        pltpu.make_async_copy(
            table_hbm.at[idx_v.at[pl.ds(c * _C, _C)]], rows[b],
            gsems[b]).wait()

    def write(c, b):
        pltpu.async_copy(rows[b], out_hbm.at[pl.ds(base + c * _C, _C)],
                         wsems[b])

    def wait_write(c, b):
        pltpu.make_async_copy(
            rows[b], out_hbm.at[pl.ds(base + c * _C, _C)], wsems[b]).wait()

    for c in range(_NBUF):
        gather(c, c % _NBUF)

    @pl.loop(0, _NCHUNK - _NBUF, step=_NBUF)
    def _steady(c0):
        for j in range(_NBUF):
            c = c0 + j
            wait_gather(c, j)
            write(c, j)
            wait_write(c, j)
            gather(c + _NBUF, j)

    for c in range(_NCHUNK - _NBUF, _NCHUNK):
        wait_gather(c, c % _NBUF)
        write(c, c % _NBUF)
    for c in range(_NCHUNK - _NBUF, _NCHUNK):
        wait_write(c, c % _NBUF)


@functools.lru_cache(maxsize=None)
def _sc_gather_fn():
    return pl.kernel(
        _gather_body,
        out_type=jax.ShapeDtypeStruct((_B, D_MODEL), jnp.float32),
        mesh=plsc.VectorSubcoreMesh(
            core_axis_name="c", subcore_axis_name="s",
            num_cores=_NC, num_subcores=_NS),
        scratch_types=[
            pltpu.VMEM((_BPW,), jnp.int32),
            [pltpu.VMEM((_C, D_MODEL), jnp.float32) for _ in range(_NBUF)],
            [pltpu.SemaphoreType.DMA for _ in range(_NBUF)],
            [pltpu.SemaphoreType.DMA for _ in range(_NBUF)],
        ],
    )


def kernel(tokens, W_E):
    table = jnp.transpose(W_E)
    out = _sc_gather_fn()(table, tokens.astype(jnp.int32))
    return out.reshape(tokens.shape[0], tokens.shape[1], D_MODEL)
